# baseline (speedup 1.0000x reference)
import functools

import jax
import jax.numpy as jnp
from jax import lax
from jax.experimental import pallas as pl
from jax.experimental.pallas import tpu as pltpu
from jax.experimental.pallas import tpu_sc as plsc

_B = 16
_C = 192
_S = 56
_OUT_HW = 2 * _S
_NW = 32
_PER_W = _B * _C // _NW


def _body(in_hbm, out_hbm, in_v0, in_v1, out_v0, out_v1, si0, si1, so0, so1):
    wid = lax.axis_index("s") * 2 + lax.axis_index("c")
    b = wid & 15
    c0 = (wid >> 4) * _PER_W
    lane = lax.iota(jnp.int32, 16)
    half = lane >> 1
    par = lane & 1

    in_bufs = (in_v0, in_v1)
    out_bufs = (out_v0, out_v1)
    sin = (si0, si1)
    sout = (so0, so1)

    def in_cp(g, s):
        return pltpu.make_async_copy(
            in_hbm.at[b, pl.ds(4 * (c0 + g), 4)], in_bufs[s], sin[s])

    def out_cp(g, s):
        return pltpu.make_async_copy(
            out_bufs[s], out_hbm.at[b, c0 + g], sout[s])

    def compute(s):
        iv, ov = in_bufs[s], out_bufs[s]

        @plsc.parallel_loop(0, _S, unroll=8)
        def per_row(j):
            jj = jnp.full((16,), j, jnp.int32)
            for dy in (0, 1):
                qv = 2 * par + dy
                for v in range(7):
                    ii = 8 * v + half
                    ov[2 * j + dy, pl.ds(16 * v, 16)] = (
                        plsc.load_gather(iv, [qv, jj, ii]))

    in_cp(0, 0).start()
    in_cp(1, 1).start()
    for g in (0, 1):
        in_cp(g, g).wait()
        compute(g)
        out_cp(g, g).start()
        in_cp(g + 2, g).start()

    def steady(g2, carry):
        for s in (0, 1):
            g = 2 * g2 + s
            in_cp(g, s).wait()
            out_cp(g - 2, s).wait()
            compute(s)
            out_cp(g, s).start()
            in_cp(g + 2, s).start()
        return carry

    lax.fori_loop(1, _PER_W // 2 - 1, steady, 0)

    for g in (_PER_W - 2, _PER_W - 1):
        s = g & 1
        in_cp(g, s).wait()
        out_cp(g - 2, s).wait()
        compute(s)
        out_cp(g, s).start()
    out_cp(_PER_W - 2, 0).wait()
    out_cp(_PER_W - 1, 1).wait()


@jax.jit
def _shuffle(x):
    mesh = plsc.VectorSubcoreMesh(core_axis_name="c", subcore_axis_name="s")
    f = functools.partial(
        pl.kernel,
        mesh=mesh,
        out_type=jax.ShapeDtypeStruct((_B, _C, _OUT_HW, _OUT_HW), jnp.float32),
        scratch_types=[
            pltpu.VMEM((4, _S, _S), jnp.float32),
            pltpu.VMEM((4, _S, _S), jnp.float32),
            pltpu.VMEM((_OUT_HW, _OUT_HW), jnp.float32),
            pltpu.VMEM((_OUT_HW, _OUT_HW), jnp.float32),
            pltpu.SemaphoreType.DMA,
            pltpu.SemaphoreType.DMA,
            pltpu.SemaphoreType.DMA,
            pltpu.SemaphoreType.DMA,
        ],
        compiler_params=pltpu.CompilerParams(
            needs_layout_passes=False, use_tc_tiling_on_sc=True),
    )(_body)
    return f(x)


def kernel(input, idx_c, idx_y, idx_x):
    del idx_c, idx_y, idx_x
    return _shuffle(input)


# speedup vs baseline: 2.3869x; 2.3869x over previous
import functools

import jax
import jax.numpy as jnp
from jax import lax
from jax.experimental import pallas as pl
from jax.experimental.pallas import tpu as pltpu
from jax.experimental.pallas import tpu_sc as plsc

_B = 16
_C = 192
_S = 56
_OUT_HW = 2 * _S
_CH = 4 * _C
_HCH = _CH // 2
_HC = _C // 2


def _body(in_hbm, out_hbm, in_v0, in_v1, out_v0, out_v1, si0, si1, so0, so1):
    wid = lax.axis_index("s") * 2 + lax.axis_index("c")
    b = wid & 15
    h = wid >> 4
    ch0 = h * _HCH
    c0 = h * _HC
    lane = lax.iota(jnp.int32, 16)
    half = lane >> 1
    par2 = 2 * (lane & 1)

    in_bufs = (in_v0, in_v1)
    out_bufs = (out_v0, out_v1)
    sin = (si0, si1)
    sout = (so0, so1)

    def in_cp(j, s):
        return pltpu.make_async_copy(
            in_hbm.at[b, j, :, pl.ds(ch0, _HCH)], in_bufs[s], sin[s])

    def out_cp(j, s):
        return pltpu.make_async_copy(
            out_bufs[s], out_hbm.at[b, pl.ds(c0, _HC), pl.ds(2 * j, 2)],
            sout[s])

    def compute(s):
        iv, ov = in_bufs[s], out_bufs[s]

        @plsc.parallel_loop(0, _HC, unroll=8)
        def per_c(c):
            for dy in (0, 1):
                cc = par2 + (4 * c + dy)
                for v in range(7):
                    ii = 8 * v + half
                    ov[c, dy, pl.ds(16 * v, 16)] = (
                        plsc.load_gather(iv, [ii, cc]))

    in_cp(0, 0).start()
    in_cp(1, 1).start()
    for j in (0, 1):
        in_cp(j, j).wait()
        compute(j)
        out_cp(j, j).start()
        in_cp(j + 2, j).start()

    def steady(j2, carry):
        for s in (0, 1):
            j = 2 * j2 + s
            in_cp(j, s).wait()
            out_cp(j - 2, s).wait()
            compute(s)
            out_cp(j, s).start()
            in_cp(j + 2, s).start()
        return carry

    lax.fori_loop(1, _S // 2 - 1, steady, 0)

    for j in (_S - 2, _S - 1):
        s = j & 1
        in_cp(j, s).wait()
        out_cp(j - 2, s).wait()
        compute(s)
        out_cp(j, s).start()
    out_cp(_S - 2, 0).wait()
    out_cp(_S - 1, 1).wait()


@jax.jit
def _shuffle(x):
    x_t = jnp.transpose(x, (0, 2, 3, 1))
    mesh = plsc.VectorSubcoreMesh(core_axis_name="c", subcore_axis_name="s")
    f = functools.partial(
        pl.kernel,
        mesh=mesh,
        out_type=jax.ShapeDtypeStruct((_B, _C, _OUT_HW, _OUT_HW), jnp.float32),
        scratch_types=[
            pltpu.VMEM((_S, _HCH), jnp.float32),
            pltpu.VMEM((_S, _HCH), jnp.float32),
            pltpu.VMEM((_HC, 2, _OUT_HW), jnp.float32),
            pltpu.VMEM((_HC, 2, _OUT_HW), jnp.float32),
            pltpu.SemaphoreType.DMA,
            pltpu.SemaphoreType.DMA,
            pltpu.SemaphoreType.DMA,
            pltpu.SemaphoreType.DMA,
        ],
        compiler_params=pltpu.CompilerParams(needs_layout_passes=False),
    )(_body)
    return f(x_t)


def kernel(input, idx_c, idx_y, idx_x):
    del idx_c, idx_y, idx_x
    return _shuffle(input)
